# baseline (speedup 1.0000x reference)
import functools

import jax
import jax.numpy as jnp
from jax.experimental import pallas as pl
from jax.experimental.pallas import tpu as pltpu

_EMB = 1024
_HEADS = 16
_T = 2048
_FF = 4

_BQ = 512
_BK = 512
_NQ = _T // _BQ
_NK = _T // _BK

_VMEM_LIMIT = 60 * 1024 * 1024


def _proj_body(x_ref, w_ref, o_ref, *, scale):
    w = w_ref[...].astype(jnp.bfloat16)
    o = jax.lax.dot(x_ref[...], w, preferred_element_type=jnp.float32)
    if scale != 1.0:
        o = o * scale
    o_ref[0] = o.astype(jnp.bfloat16)


def _layernorm(t, g, b):
    m = jnp.mean(t, axis=1, keepdims=True)
    c = t - m
    v = jnp.mean(c * c, axis=1, keepdims=True)
    return c * jax.lax.rsqrt(v + 1e-5) * g + b


_BQ2 = 2 * _BQ


def _attn_proj_body(
    q_ref, k_ref, v_ref, wu_ref, x_ref, bu_ref, g_ref, be_ref,
    out_ref, acc_ref, m_ref, l_ref, pacc_ref,
):
    h = pl.program_id(0)
    j = pl.program_id(1)
    ik = pl.program_id(2)
    top = 2 * j + 1

    @pl.when(ik == 0)
    def _init():
        m_ref[...] = jnp.full_like(m_ref, -jnp.inf)
        l_ref[...] = jnp.zeros_like(l_ref)
        acc_ref[...] = jnp.zeros_like(acc_ref)

    @pl.when(ik <= top)
    def _update():
        q = q_ref[0]
        k = k_ref[0]
        s = jax.lax.dot_general(
            q, k, (((1,), (1,)), ((), ())),
            preferred_element_type=jnp.float32,
        )
        rows = j * _BQ2 + jax.lax.broadcasted_iota(jnp.int32, (_BQ2, _BK), 0)
        cols = ik * _BK + jax.lax.broadcasted_iota(jnp.int32, (_BQ2, _BK), 1)
        s = jnp.where(cols <= rows, s, -jnp.inf)
        m_prev = m_ref[...]
        m_new = jnp.maximum(m_prev, jnp.max(s, axis=1, keepdims=True))
        alpha = jnp.exp(m_prev - m_new)
        p = jnp.exp(s - m_new)
        l_ref[...] = l_ref[...] * alpha + jnp.sum(p, axis=1, keepdims=True)
        acc_ref[...] = acc_ref[...] * alpha + jax.lax.dot(
            p.astype(jnp.bfloat16), v_ref[0],
            preferred_element_type=jnp.float32,
        )
        m_ref[...] = m_new

    @pl.when(ik == top)
    def _proj():
        o = (acc_ref[...] / l_ref[...]).astype(jnp.bfloat16)
        wu = wu_ref[...].astype(jnp.bfloat16)
        part = jax.lax.dot(o, wu, preferred_element_type=jnp.float32)
        base = pl.multiple_of(j * _BQ2, _BQ2)
        prev = pacc_ref[pl.ds(base, _BQ2), :]
        pacc_ref[pl.ds(base, _BQ2), :] = jnp.where(h == 0, part, prev + part)

        @pl.when(h == _HEADS - 1)
        def _finish():
            t = pacc_ref[pl.ds(base, _BQ2), :] + bu_ref[...] + x_ref[...]
            out_ref[pl.ds(base, _BQ2), :] = _layernorm(
                t, g_ref[...], be_ref[...]
            )


def _ff_ln_body(x_ref, w1_ref, b1_ref, w2_ref, b2_ref, g_ref, be_ref, out_ref):
    x = x_ref[...]
    w1 = w1_ref[...].astype(jnp.bfloat16)
    hid = jax.lax.dot(
        x.astype(jnp.bfloat16), w1, preferred_element_type=jnp.float32
    )
    hid = jnp.maximum(hid + b1_ref[...], 0.0)
    w2 = w2_ref[...].astype(jnp.bfloat16)
    f = jax.lax.dot(
        hid.astype(jnp.bfloat16), w2, preferred_element_type=jnp.float32
    )
    t = f + b2_ref[...] + x
    out_ref[...] = _layernorm(t, g_ref[...], be_ref[...])


def kernel(x, Wq, Wk, Wv, Wu, bu, g1, be1, g2, be2, W1, b1, W2, b2):
    b, t, e = x.shape
    x2d = x.reshape(t, e)
    xb = x2d.astype(jnp.bfloat16)

    cp = lambda sem: pltpu.CompilerParams(
        dimension_semantics=sem, vmem_limit_bytes=_VMEM_LIMIT
    )

    def proj(w, scale):
        return pl.pallas_call(
            functools.partial(_proj_body, scale=scale),
            grid=(_HEADS,),
            in_specs=[
                pl.BlockSpec((_T, _EMB), lambda h: (0, 0)),
                pl.BlockSpec((_EMB, _EMB), lambda h: (0, h)),
            ],
            out_specs=pl.BlockSpec((1, _T, _EMB), lambda h: (h, 0, 0)),
            out_shape=jax.ShapeDtypeStruct((_HEADS, _T, _EMB), jnp.bfloat16),
            compiler_params=cp(("arbitrary",)),
        )(xb, w)

    q = proj(Wq, 1.0 / 32.0)
    k = proj(Wk, 1.0)
    v = proj(Wv, 1.0)

    x1 = pl.pallas_call(
        _attn_proj_body,
        grid=(_HEADS, _NQ // 2, _NK),
        in_specs=[
            pl.BlockSpec((1, 2 * _BQ, _EMB), lambda h, j, ik: (h, j, 0)),
            pl.BlockSpec(
                (1, _BK, _EMB),
                lambda h, j, ik: (h, jnp.minimum(ik, 2 * j + 1), 0),
            ),
            pl.BlockSpec(
                (1, _BK, _EMB),
                lambda h, j, ik: (h, jnp.minimum(ik, 2 * j + 1), 0),
            ),
            pl.BlockSpec((_EMB, _EMB), lambda h, j, ik: (h, 0)),
            pl.BlockSpec((2 * _BQ, _EMB), lambda h, j, ik: (j, 0)),
            pl.BlockSpec((1, _EMB), lambda h, j, ik: (0, 0)),
            pl.BlockSpec((1, _EMB), lambda h, j, ik: (0, 0)),
            pl.BlockSpec((1, _EMB), lambda h, j, ik: (0, 0)),
        ],
        out_specs=pl.BlockSpec((_T, _EMB), lambda h, j, ik: (0, 0)),
        out_shape=jax.ShapeDtypeStruct((_T, _EMB), jnp.float32),
        scratch_shapes=[
            pltpu.VMEM((_BQ2, _EMB), jnp.float32),
            pltpu.VMEM((_BQ2, 1), jnp.float32),
            pltpu.VMEM((_BQ2, 1), jnp.float32),
            pltpu.VMEM((_T, _EMB), jnp.float32),
        ],
        compiler_params=cp(("arbitrary", "arbitrary", "arbitrary")),
    )(
        q,
        k,
        v,
        Wu,
        x2d,
        bu.reshape(1, _EMB),
        g1.reshape(1, _EMB),
        be1.reshape(1, _EMB),
    )

    br = 512
    x2 = pl.pallas_call(
        _ff_ln_body,
        grid=(_T // br,),
        in_specs=[
            pl.BlockSpec((br, _EMB), lambda i: (i, 0)),
            pl.BlockSpec((_EMB, _FF * _EMB), lambda i: (0, 0)),
            pl.BlockSpec((1, _FF * _EMB), lambda i: (0, 0)),
            pl.BlockSpec((_FF * _EMB, _EMB), lambda i: (0, 0)),
            pl.BlockSpec((1, _EMB), lambda i: (0, 0)),
            pl.BlockSpec((1, _EMB), lambda i: (0, 0)),
            pl.BlockSpec((1, _EMB), lambda i: (0, 0)),
        ],
        out_specs=pl.BlockSpec((br, _EMB), lambda i: (i, 0)),
        out_shape=jax.ShapeDtypeStruct((_T, _EMB), jnp.float32),
        compiler_params=cp(("arbitrary",)),
    )(
        x1,
        W1,
        b1.reshape(1, _FF * _EMB),
        W2,
        b2.reshape(1, _EMB),
        g2.reshape(1, _EMB),
        be2.reshape(1, _EMB),
    )

    return x2.reshape(b, t, e)


# speedup vs baseline: 1.0843x; 1.0843x over previous
import functools

import jax
import jax.numpy as jnp
from jax.experimental import pallas as pl
from jax.experimental.pallas import tpu as pltpu

_EMB = 1024
_HEADS = 16
_T = 2048
_FF = 4

_BQ = 512
_BK = 512
_NQ = _T // _BQ
_NK = _T // _BK

_VMEM_LIMIT = 60 * 1024 * 1024


def _proj_body(x_ref, w_ref, o_ref, *, scale):
    w = w_ref[...].astype(jnp.bfloat16)
    o = jax.lax.dot(x_ref[...], w, preferred_element_type=jnp.float32)
    if scale != 1.0:
        o = o * scale
    o_ref[0] = o.astype(jnp.bfloat16)


def _layernorm(t, g, b):
    m = jnp.mean(t, axis=1, keepdims=True)
    c = t - m
    v = jnp.mean(c * c, axis=1, keepdims=True)
    return c * jax.lax.rsqrt(v + 1e-5) * g + b


def _attn_proj_body(
    q_ref, k_ref, v_ref, wu_ref, x_ref, bu_ref, g_ref, be_ref,
    out_ref, acc0_ref, m0_ref, l0_ref, acc1_ref, m1_ref, l1_ref, pacc_ref,
):
    h = pl.program_id(0)
    j = pl.program_id(1)
    ik = pl.program_id(2)

    chains = ((0, acc0_ref, m0_ref, l0_ref), (1, acc1_ref, m1_ref, l1_ref))
    for c, acc_ref, m_ref, l_ref in chains:
        iq = 2 * j + c

        @pl.when(ik == 0)
        def _init(acc_ref=acc_ref, m_ref=m_ref, l_ref=l_ref):
            m_ref[...] = jnp.full_like(m_ref, -jnp.inf)
            l_ref[...] = jnp.zeros_like(l_ref)
            acc_ref[...] = jnp.zeros_like(acc_ref)

        @pl.when(ik <= iq)
        def _update(c=c, iq=iq, acc_ref=acc_ref, m_ref=m_ref, l_ref=l_ref):
            q = q_ref[0, c * _BQ:(c + 1) * _BQ, :]
            k = k_ref[0]
            st = jax.lax.dot_general(
                k, q, (((1,), (1,)), ((), ())),
                preferred_element_type=jnp.float32,
            )
            keys = ik * _BK + jax.lax.broadcasted_iota(jnp.int32, (_BK, _BQ), 0)
            rows = iq * _BQ + jax.lax.broadcasted_iota(jnp.int32, (_BK, _BQ), 1)
            st = jnp.where(keys <= rows, st, -jnp.inf)
            m_prev = m_ref[...]
            m_new = jnp.maximum(m_prev, jnp.max(st, axis=0, keepdims=True))
            alpha = jnp.exp(m_prev - m_new)
            pt = jnp.exp(st - m_new)
            l_ref[...] = l_ref[...] * alpha + jnp.sum(pt, axis=0, keepdims=True)
            acc_ref[...] = acc_ref[...] * alpha + jax.lax.dot_general(
                v_ref[0], pt.astype(jnp.bfloat16),
                (((0,), (0,)), ((), ())),
                preferred_element_type=jnp.float32,
            )
            m_ref[...] = m_new

        @pl.when(ik == iq)
        def _proj(c=c, iq=iq, acc_ref=acc_ref, l_ref=l_ref):
            o = (acc_ref[...] / l_ref[...]).astype(jnp.bfloat16)
            wu = wu_ref[...].astype(jnp.bfloat16)
            part = jax.lax.dot_general(
                o, wu, (((0,), (0,)), ((), ())),
                preferred_element_type=jnp.float32,
            )
            base = pl.multiple_of(iq * _BQ, _BQ)
            prev = pacc_ref[pl.ds(base, _BQ), :]
            pacc_ref[pl.ds(base, _BQ), :] = jnp.where(h == 0, part, prev + part)

            @pl.when(h == _HEADS - 1)
            def _finish():
                t = (
                    pacc_ref[pl.ds(base, _BQ), :]
                    + bu_ref[...]
                    + x_ref[c * _BQ:(c + 1) * _BQ, :]
                )
                out_ref[pl.ds(base, _BQ), :] = _layernorm(
                    t, g_ref[...], be_ref[...]
                )


def _ff_ln_body(x_ref, w1_ref, b1_ref, w2_ref, b2_ref, g_ref, be_ref, out_ref):
    x = x_ref[...]
    w1 = w1_ref[...].astype(jnp.bfloat16)
    hid = jax.lax.dot(
        x.astype(jnp.bfloat16), w1, preferred_element_type=jnp.float32
    )
    hid = jnp.maximum(hid + b1_ref[...], 0.0)
    w2 = w2_ref[...].astype(jnp.bfloat16)
    f = jax.lax.dot(
        hid.astype(jnp.bfloat16), w2, preferred_element_type=jnp.float32
    )
    t = f + b2_ref[...] + x
    out_ref[...] = _layernorm(t, g_ref[...], be_ref[...])


def kernel(x, Wq, Wk, Wv, Wu, bu, g1, be1, g2, be2, W1, b1, W2, b2):
    b, t, e = x.shape
    x2d = x.reshape(t, e)
    xb = x2d.astype(jnp.bfloat16)

    cp = lambda sem: pltpu.CompilerParams(
        dimension_semantics=sem, vmem_limit_bytes=_VMEM_LIMIT
    )

    def proj(w, scale):
        return pl.pallas_call(
            functools.partial(_proj_body, scale=scale),
            grid=(_HEADS,),
            in_specs=[
                pl.BlockSpec((_T, _EMB), lambda h: (0, 0)),
                pl.BlockSpec((_EMB, _EMB), lambda h: (0, h)),
            ],
            out_specs=pl.BlockSpec((1, _T, _EMB), lambda h: (h, 0, 0)),
            out_shape=jax.ShapeDtypeStruct((_HEADS, _T, _EMB), jnp.bfloat16),
            compiler_params=cp(("arbitrary",)),
        )(xb, w)

    q = proj(Wq, 1.0 / 32.0)
    k = proj(Wk, 1.0)
    v = proj(Wv, 1.0)

    x1 = pl.pallas_call(
        _attn_proj_body,
        grid=(_HEADS, _NQ // 2, _NK),
        in_specs=[
            pl.BlockSpec((1, 2 * _BQ, _EMB), lambda h, j, ik: (h, j, 0)),
            pl.BlockSpec(
                (1, _BK, _EMB),
                lambda h, j, ik: (h, jnp.minimum(ik, 2 * j + 1), 0),
            ),
            pl.BlockSpec(
                (1, _BK, _EMB),
                lambda h, j, ik: (h, jnp.minimum(ik, 2 * j + 1), 0),
            ),
            pl.BlockSpec((_EMB, _EMB), lambda h, j, ik: (h, 0)),
            pl.BlockSpec((2 * _BQ, _EMB), lambda h, j, ik: (j, 0)),
            pl.BlockSpec((1, _EMB), lambda h, j, ik: (0, 0)),
            pl.BlockSpec((1, _EMB), lambda h, j, ik: (0, 0)),
            pl.BlockSpec((1, _EMB), lambda h, j, ik: (0, 0)),
        ],
        out_specs=pl.BlockSpec((_T, _EMB), lambda h, j, ik: (0, 0)),
        out_shape=jax.ShapeDtypeStruct((_T, _EMB), jnp.float32),
        scratch_shapes=[
            pltpu.VMEM((_EMB, _BQ), jnp.float32),
            pltpu.VMEM((1, _BQ), jnp.float32),
            pltpu.VMEM((1, _BQ), jnp.float32),
            pltpu.VMEM((_EMB, _BQ), jnp.float32),
            pltpu.VMEM((1, _BQ), jnp.float32),
            pltpu.VMEM((1, _BQ), jnp.float32),
            pltpu.VMEM((_T, _EMB), jnp.float32),
        ],
        compiler_params=cp(("arbitrary", "arbitrary", "arbitrary")),
    )(
        q,
        k,
        v,
        Wu,
        x2d,
        bu.reshape(1, _EMB),
        g1.reshape(1, _EMB),
        be1.reshape(1, _EMB),
    )

    br = 512
    x2 = pl.pallas_call(
        _ff_ln_body,
        grid=(_T // br,),
        in_specs=[
            pl.BlockSpec((br, _EMB), lambda i: (i, 0)),
            pl.BlockSpec((_EMB, _FF * _EMB), lambda i: (0, 0)),
            pl.BlockSpec((1, _FF * _EMB), lambda i: (0, 0)),
            pl.BlockSpec((_FF * _EMB, _EMB), lambda i: (0, 0)),
            pl.BlockSpec((1, _EMB), lambda i: (0, 0)),
            pl.BlockSpec((1, _EMB), lambda i: (0, 0)),
            pl.BlockSpec((1, _EMB), lambda i: (0, 0)),
        ],
        out_specs=pl.BlockSpec((br, _EMB), lambda i: (i, 0)),
        out_shape=jax.ShapeDtypeStruct((_T, _EMB), jnp.float32),
        compiler_params=cp(("arbitrary",)),
    )(
        x1,
        W1,
        b1.reshape(1, _FF * _EMB),
        W2,
        b2.reshape(1, _EMB),
        g2.reshape(1, _EMB),
        be2.reshape(1, _EMB),
    )

    return x2.reshape(b, t, e)


# speedup vs baseline: 1.1243x; 1.0369x over previous
import functools

import jax
import jax.numpy as jnp
from jax.experimental import pallas as pl
from jax.experimental.pallas import tpu as pltpu

_EMB = 1024
_HEADS = 16
_T = 2048
_FF = 4

_BQ = 512
_BK = 512
_NQ = _T // _BQ
_NK = _T // _BK

_VMEM_LIMIT = 60 * 1024 * 1024


def _qkv_body(x_ref, wq_ref, wk_ref, wv_ref, q_ref, k_ref, v_ref):
    x = x_ref[...]
    wq = wq_ref[...].astype(jnp.bfloat16)
    qo = jax.lax.dot(x, wq, preferred_element_type=jnp.float32)
    q_ref[0] = (qo * (1.0 / 32.0)).astype(jnp.bfloat16)
    wk = wk_ref[...].astype(jnp.bfloat16)
    k_ref[0] = jax.lax.dot(
        x, wk, preferred_element_type=jnp.float32
    ).astype(jnp.bfloat16)
    wv = wv_ref[...].astype(jnp.bfloat16)
    v_ref[0] = jax.lax.dot(
        x, wv, preferred_element_type=jnp.float32
    ).astype(jnp.bfloat16)


def _layernorm(t, g, b):
    m = jnp.mean(t, axis=1, keepdims=True)
    c = t - m
    v = jnp.mean(c * c, axis=1, keepdims=True)
    return c * jax.lax.rsqrt(v + 1e-5) * g + b


def _attn_proj_body(
    q_ref, k_ref, v_ref, wu_ref, x_ref, bu_ref, g_ref, be_ref,
    out_ref, acc0_ref, m0_ref, l0_ref, acc1_ref, m1_ref, l1_ref, pacc_ref,
):
    h = pl.program_id(0)
    j = pl.program_id(1)
    ik = pl.program_id(2)

    chains = ((0, acc0_ref, m0_ref, l0_ref), (1, acc1_ref, m1_ref, l1_ref))
    for c, acc_ref, m_ref, l_ref in chains:
        iq = 2 * j + c

        @pl.when(ik == 0)
        def _init(acc_ref=acc_ref, m_ref=m_ref, l_ref=l_ref):
            m_ref[...] = jnp.full_like(m_ref, -jnp.inf)
            l_ref[...] = jnp.zeros_like(l_ref)
            acc_ref[...] = jnp.zeros_like(acc_ref)

        def _flash_update(masked, c, iq, acc_ref, m_ref, l_ref):
            q = q_ref[0, c * _BQ:(c + 1) * _BQ, :]
            k = k_ref[0]
            st = jax.lax.dot_general(
                k, q, (((1,), (1,)), ((), ())),
                preferred_element_type=jnp.float32,
            )
            if masked:
                keys = jax.lax.broadcasted_iota(jnp.int32, (_BK, _BQ), 0)
                rows = jax.lax.broadcasted_iota(jnp.int32, (_BK, _BQ), 1)
                st = jnp.where(keys <= rows, st, -jnp.inf)
            m_prev = m_ref[...]
            m_new = jnp.maximum(m_prev, jnp.max(st, axis=0, keepdims=True))
            alpha = jnp.exp(m_prev - m_new)
            pt = jnp.exp(st - m_new)
            l_ref[...] = l_ref[...] * alpha + jnp.sum(pt, axis=0, keepdims=True)
            acc_ref[...] = acc_ref[...] * alpha + jax.lax.dot_general(
                v_ref[0], pt.astype(jnp.bfloat16),
                (((0,), (0,)), ((), ())),
                preferred_element_type=jnp.float32,
            )
            m_ref[...] = m_new

        @pl.when(ik < iq)
        def _update(c=c, iq=iq, acc_ref=acc_ref, m_ref=m_ref, l_ref=l_ref):
            _flash_update(False, c, iq, acc_ref, m_ref, l_ref)

        @pl.when(ik == iq)
        def _update_diag(c=c, iq=iq, acc_ref=acc_ref, m_ref=m_ref, l_ref=l_ref):
            _flash_update(True, c, iq, acc_ref, m_ref, l_ref)

        @pl.when(ik == iq)
        def _proj(c=c, iq=iq, acc_ref=acc_ref, l_ref=l_ref):
            o = (acc_ref[...] / l_ref[...]).astype(jnp.bfloat16)
            wu = wu_ref[...].astype(jnp.bfloat16)
            part = jax.lax.dot_general(
                o, wu, (((0,), (0,)), ((), ())),
                preferred_element_type=jnp.float32,
            )
            base = pl.multiple_of(iq * _BQ, _BQ)
            prev = pacc_ref[pl.ds(base, _BQ), :]
            pacc_ref[pl.ds(base, _BQ), :] = jnp.where(h == 0, part, prev + part)

            @pl.when(h == _HEADS - 1)
            def _finish():
                t = (
                    pacc_ref[pl.ds(base, _BQ), :]
                    + bu_ref[...]
                    + x_ref[c * _BQ:(c + 1) * _BQ, :]
                )
                out_ref[pl.ds(base, _BQ), :] = _layernorm(
                    t, g_ref[...], be_ref[...]
                )


def _ff_ln_body(x_ref, w1_ref, b1_ref, w2_ref, b2_ref, g_ref, be_ref, out_ref):
    x = x_ref[...]
    w1 = w1_ref[...].astype(jnp.bfloat16)
    hid = jax.lax.dot(
        x.astype(jnp.bfloat16), w1, preferred_element_type=jnp.float32
    )
    hid = jnp.maximum(hid + b1_ref[...], 0.0)
    w2 = w2_ref[...].astype(jnp.bfloat16)
    f = jax.lax.dot(
        hid.astype(jnp.bfloat16), w2, preferred_element_type=jnp.float32
    )
    t = f + b2_ref[...] + x
    out_ref[...] = _layernorm(t, g_ref[...], be_ref[...])


def kernel(x, Wq, Wk, Wv, Wu, bu, g1, be1, g2, be2, W1, b1, W2, b2):
    b, t, e = x.shape
    x2d = x.reshape(t, e)
    xb = x2d.astype(jnp.bfloat16)

    cp = lambda sem: pltpu.CompilerParams(
        dimension_semantics=sem, vmem_limit_bytes=_VMEM_LIMIT
    )

    q, k, v = pl.pallas_call(
        _qkv_body,
        grid=(_HEADS,),
        in_specs=[
            pl.BlockSpec((_T, _EMB), lambda h: (0, 0)),
            pl.BlockSpec((_EMB, _EMB), lambda h: (0, h)),
            pl.BlockSpec((_EMB, _EMB), lambda h: (0, h)),
            pl.BlockSpec((_EMB, _EMB), lambda h: (0, h)),
        ],
        out_specs=[
            pl.BlockSpec((1, _T, _EMB), lambda h: (h, 0, 0)),
            pl.BlockSpec((1, _T, _EMB), lambda h: (h, 0, 0)),
            pl.BlockSpec((1, _T, _EMB), lambda h: (h, 0, 0)),
        ],
        out_shape=[
            jax.ShapeDtypeStruct((_HEADS, _T, _EMB), jnp.bfloat16),
            jax.ShapeDtypeStruct((_HEADS, _T, _EMB), jnp.bfloat16),
            jax.ShapeDtypeStruct((_HEADS, _T, _EMB), jnp.bfloat16),
        ],
        compiler_params=cp(("arbitrary",)),
    )(xb, Wq, Wk, Wv)

    x1 = pl.pallas_call(
        _attn_proj_body,
        grid=(_HEADS, _NQ // 2, _NK),
        in_specs=[
            pl.BlockSpec((1, 2 * _BQ, _EMB), lambda h, j, ik: (h, j, 0)),
            pl.BlockSpec(
                (1, _BK, _EMB),
                lambda h, j, ik: (h, jnp.minimum(ik, 2 * j + 1), 0),
            ),
            pl.BlockSpec(
                (1, _BK, _EMB),
                lambda h, j, ik: (h, jnp.minimum(ik, 2 * j + 1), 0),
            ),
            pl.BlockSpec((_EMB, _EMB), lambda h, j, ik: (h, 0)),
            pl.BlockSpec((2 * _BQ, _EMB), lambda h, j, ik: (j, 0)),
            pl.BlockSpec((1, _EMB), lambda h, j, ik: (0, 0)),
            pl.BlockSpec((1, _EMB), lambda h, j, ik: (0, 0)),
            pl.BlockSpec((1, _EMB), lambda h, j, ik: (0, 0)),
        ],
        out_specs=pl.BlockSpec((_T, _EMB), lambda h, j, ik: (0, 0)),
        out_shape=jax.ShapeDtypeStruct((_T, _EMB), jnp.float32),
        scratch_shapes=[
            pltpu.VMEM((_EMB, _BQ), jnp.float32),
            pltpu.VMEM((1, _BQ), jnp.float32),
            pltpu.VMEM((1, _BQ), jnp.float32),
            pltpu.VMEM((_EMB, _BQ), jnp.float32),
            pltpu.VMEM((1, _BQ), jnp.float32),
            pltpu.VMEM((1, _BQ), jnp.float32),
            pltpu.VMEM((_T, _EMB), jnp.float32),
        ],
        compiler_params=cp(("arbitrary", "arbitrary", "arbitrary")),
    )(
        q,
        k,
        v,
        Wu,
        x2d,
        bu.reshape(1, _EMB),
        g1.reshape(1, _EMB),
        be1.reshape(1, _EMB),
    )

    br = 512
    x2 = pl.pallas_call(
        _ff_ln_body,
        grid=(_T // br,),
        in_specs=[
            pl.BlockSpec((br, _EMB), lambda i: (i, 0)),
            pl.BlockSpec((_EMB, _FF * _EMB), lambda i: (0, 0)),
            pl.BlockSpec((1, _FF * _EMB), lambda i: (0, 0)),
            pl.BlockSpec((_FF * _EMB, _EMB), lambda i: (0, 0)),
            pl.BlockSpec((1, _EMB), lambda i: (0, 0)),
            pl.BlockSpec((1, _EMB), lambda i: (0, 0)),
            pl.BlockSpec((1, _EMB), lambda i: (0, 0)),
        ],
        out_specs=pl.BlockSpec((br, _EMB), lambda i: (i, 0)),
        out_shape=jax.ShapeDtypeStruct((_T, _EMB), jnp.float32),
        compiler_params=cp(("arbitrary",)),
    )(
        x1,
        W1,
        b1.reshape(1, _FF * _EMB),
        W2,
        b2.reshape(1, _EMB),
        g2.reshape(1, _EMB),
        be2.reshape(1, _EMB),
    )

    return x2.reshape(b, t, e)


# speedup vs baseline: 1.2095x; 1.0758x over previous
import functools

import jax
import jax.numpy as jnp
from jax.experimental import pallas as pl
from jax.experimental.pallas import tpu as pltpu

_EMB = 1024
_HEADS = 16
_T = 2048
_FF = 4

_BQ = 512
_BK = 1024
_NQ = _T // _BQ
_NK = _T // _BK

_VMEM_LIMIT = 60 * 1024 * 1024


def _qkv_body(x_ref, wq_ref, wk_ref, wv_ref, q_ref, k_ref, v_ref):
    x = x_ref[...]
    wq = wq_ref[...].astype(jnp.bfloat16)
    qo = jax.lax.dot(x, wq, preferred_element_type=jnp.float32)
    q_ref[0] = (qo * (1.0 / 32.0)).astype(jnp.bfloat16)
    wk = wk_ref[...].astype(jnp.bfloat16)
    k_ref[0] = jax.lax.dot(
        x, wk, preferred_element_type=jnp.float32
    ).astype(jnp.bfloat16)
    wv = wv_ref[...].astype(jnp.bfloat16)
    v_ref[0] = jax.lax.dot(
        x, wv, preferred_element_type=jnp.float32
    ).astype(jnp.bfloat16)


def _layernorm(t, g, b):
    m = jnp.mean(t, axis=1, keepdims=True)
    c = t - m
    v = jnp.mean(c * c, axis=1, keepdims=True)
    return c * jax.lax.rsqrt(v + 1e-5) * g + b


def _attn_proj_body(
    q_ref, k_ref, v_ref, wu_ref, x_ref, bu_ref, g_ref, be_ref,
    out_ref, acc0_ref, m0_ref, l0_ref, acc1_ref, m1_ref, l1_ref, pacc_ref,
):
    h = pl.program_id(0)
    j = pl.program_id(1)
    kk = pl.program_id(2)

    chains = ((0, acc0_ref, m0_ref, l0_ref), (1, acc1_ref, m1_ref, l1_ref))
    for c, acc_ref, m_ref, l_ref in chains:
        iq = 2 * j + c

        @pl.when(kk == 0)
        def _init(acc_ref=acc_ref, m_ref=m_ref, l_ref=l_ref):
            m_ref[...] = jnp.full_like(m_ref, -jnp.inf)
            l_ref[...] = jnp.zeros_like(l_ref)
            acc_ref[...] = jnp.zeros_like(acc_ref)

        def _flash_update(masked, c, acc_ref, m_ref, l_ref):
            q = q_ref[0, c * _BQ:(c + 1) * _BQ, :]
            k = k_ref[0]
            st = jax.lax.dot_general(
                k, q, (((1,), (1,)), ((), ())),
                preferred_element_type=jnp.float32,
            )
            if masked:
                keys = jax.lax.broadcasted_iota(jnp.int32, (_BK, _BQ), 0)
                rows = jax.lax.broadcasted_iota(jnp.int32, (_BK, _BQ), 1)
                st = jnp.where(keys <= rows + c * _BQ, st, -jnp.inf)
            m_prev = m_ref[...]
            m_new = jnp.maximum(m_prev, jnp.max(st, axis=0, keepdims=True))
            alpha = jnp.exp(m_prev - m_new)
            pt = jnp.exp(st - m_new)
            l_ref[...] = l_ref[...] * alpha + jnp.sum(pt, axis=0, keepdims=True)
            acc_ref[...] = acc_ref[...] * alpha + jax.lax.dot_general(
                v_ref[0], pt.astype(jnp.bfloat16),
                (((0,), (0,)), ((), ())),
                preferred_element_type=jnp.float32,
            )
            m_ref[...] = m_new

        @pl.when(kk < j)
        def _update(c=c, acc_ref=acc_ref, m_ref=m_ref, l_ref=l_ref):
            _flash_update(False, c, acc_ref, m_ref, l_ref)

        @pl.when(kk == j)
        def _update_diag(c=c, acc_ref=acc_ref, m_ref=m_ref, l_ref=l_ref):
            _flash_update(True, c, acc_ref, m_ref, l_ref)

        @pl.when(kk == j)
        def _proj(c=c, iq=iq, acc_ref=acc_ref, l_ref=l_ref):
            o = (acc_ref[...] / l_ref[...]).astype(jnp.bfloat16)
            wu = wu_ref[...].astype(jnp.bfloat16)
            part = jax.lax.dot_general(
                o, wu, (((0,), (0,)), ((), ())),
                preferred_element_type=jnp.float32,
            )
            base = pl.multiple_of(iq * _BQ, _BQ)
            prev = pacc_ref[pl.ds(base, _BQ), :]
            pacc_ref[pl.ds(base, _BQ), :] = jnp.where(h == 0, part, prev + part)

            @pl.when(h == _HEADS - 1)
            def _finish():
                t = (
                    pacc_ref[pl.ds(base, _BQ), :]
                    + bu_ref[...]
                    + x_ref[c * _BQ:(c + 1) * _BQ, :]
                )
                out_ref[pl.ds(base, _BQ), :] = _layernorm(
                    t, g_ref[...], be_ref[...]
                )


def _ff_ln_body(x_ref, w1_ref, b1_ref, w2_ref, b2_ref, g_ref, be_ref, out_ref):
    x = x_ref[...]
    w1 = w1_ref[...].astype(jnp.bfloat16)
    hid = jax.lax.dot(
        x.astype(jnp.bfloat16), w1, preferred_element_type=jnp.float32
    )
    hid = jnp.maximum(hid + b1_ref[...], 0.0)
    w2 = w2_ref[...].astype(jnp.bfloat16)
    f = jax.lax.dot(
        hid.astype(jnp.bfloat16), w2, preferred_element_type=jnp.float32
    )
    t = f + b2_ref[...] + x
    out_ref[...] = _layernorm(t, g_ref[...], be_ref[...])


def kernel(x, Wq, Wk, Wv, Wu, bu, g1, be1, g2, be2, W1, b1, W2, b2):
    b, t, e = x.shape
    x2d = x.reshape(t, e)
    xb = x2d.astype(jnp.bfloat16)

    cp = lambda sem: pltpu.CompilerParams(
        dimension_semantics=sem, vmem_limit_bytes=_VMEM_LIMIT
    )

    q, k, v = pl.pallas_call(
        _qkv_body,
        grid=(_HEADS,),
        in_specs=[
            pl.BlockSpec((_T, _EMB), lambda h: (0, 0)),
            pl.BlockSpec((_EMB, _EMB), lambda h: (0, h)),
            pl.BlockSpec((_EMB, _EMB), lambda h: (0, h)),
            pl.BlockSpec((_EMB, _EMB), lambda h: (0, h)),
        ],
        out_specs=[
            pl.BlockSpec((1, _T, _EMB), lambda h: (h, 0, 0)),
            pl.BlockSpec((1, _T, _EMB), lambda h: (h, 0, 0)),
            pl.BlockSpec((1, _T, _EMB), lambda h: (h, 0, 0)),
        ],
        out_shape=[
            jax.ShapeDtypeStruct((_HEADS, _T, _EMB), jnp.bfloat16),
            jax.ShapeDtypeStruct((_HEADS, _T, _EMB), jnp.bfloat16),
            jax.ShapeDtypeStruct((_HEADS, _T, _EMB), jnp.bfloat16),
        ],
        compiler_params=cp(("arbitrary",)),
    )(xb, Wq, Wk, Wv)

    x1 = pl.pallas_call(
        _attn_proj_body,
        grid=(_HEADS, _NQ // 2, _NK),
        in_specs=[
            pl.BlockSpec((1, 2 * _BQ, _EMB), lambda h, j, kk: (h, j, 0)),
            pl.BlockSpec(
                (1, _BK, _EMB),
                lambda h, j, kk: (h, jnp.minimum(kk, j), 0),
            ),
            pl.BlockSpec(
                (1, _BK, _EMB),
                lambda h, j, kk: (h, jnp.minimum(kk, j), 0),
            ),
            pl.BlockSpec((_EMB, _EMB), lambda h, j, ik: (h, 0)),
            pl.BlockSpec((2 * _BQ, _EMB), lambda h, j, kk: (j, 0)),
            pl.BlockSpec((1, _EMB), lambda h, j, kk: (0, 0)),
            pl.BlockSpec((1, _EMB), lambda h, j, kk: (0, 0)),
            pl.BlockSpec((1, _EMB), lambda h, j, kk: (0, 0)),
        ],
        out_specs=pl.BlockSpec((_T, _EMB), lambda h, j, kk: (0, 0)),
        out_shape=jax.ShapeDtypeStruct((_T, _EMB), jnp.float32),
        scratch_shapes=[
            pltpu.VMEM((_EMB, _BQ), jnp.float32),
            pltpu.VMEM((1, _BQ), jnp.float32),
            pltpu.VMEM((1, _BQ), jnp.float32),
            pltpu.VMEM((_EMB, _BQ), jnp.float32),
            pltpu.VMEM((1, _BQ), jnp.float32),
            pltpu.VMEM((1, _BQ), jnp.float32),
            pltpu.VMEM((_T, _EMB), jnp.float32),
        ],
        compiler_params=cp(("arbitrary", "arbitrary", "arbitrary")),
    )(
        q,
        k,
        v,
        Wu,
        x2d,
        bu.reshape(1, _EMB),
        g1.reshape(1, _EMB),
        be1.reshape(1, _EMB),
    )

    br = 512
    x2 = pl.pallas_call(
        _ff_ln_body,
        grid=(_T // br,),
        in_specs=[
            pl.BlockSpec((br, _EMB), lambda i: (i, 0)),
            pl.BlockSpec((_EMB, _FF * _EMB), lambda i: (0, 0)),
            pl.BlockSpec((1, _FF * _EMB), lambda i: (0, 0)),
            pl.BlockSpec((_FF * _EMB, _EMB), lambda i: (0, 0)),
            pl.BlockSpec((1, _EMB), lambda i: (0, 0)),
            pl.BlockSpec((1, _EMB), lambda i: (0, 0)),
            pl.BlockSpec((1, _EMB), lambda i: (0, 0)),
        ],
        out_specs=pl.BlockSpec((br, _EMB), lambda i: (i, 0)),
        out_shape=jax.ShapeDtypeStruct((_T, _EMB), jnp.float32),
        compiler_params=cp(("arbitrary",)),
    )(
        x1,
        W1,
        b1.reshape(1, _FF * _EMB),
        W2,
        b2.reshape(1, _EMB),
        g2.reshape(1, _EMB),
        be2.reshape(1, _EMB),
    )

    return x2.reshape(b, t, e)
